# baseline (speedup 1.0000x reference)
import functools

import jax
import jax.numpy as jnp
from jax import lax
from jax.experimental import pallas as pl
from jax.experimental.pallas import tpu as pltpu
from jax.experimental.pallas import tpu_sc as plsc

_NC = 2
_NS = 16
_NW = _NC * _NS
_CH = 128
_NCW = 80
_NBLK = 8
_NB = _NCW // _NBLK
_IBLK = 16
_NIB = _NCW // _IBLK
_EPAD = _NW * _NCW * _CH


def _mesh():
    return plsc.VectorSubcoreMesh(
        core_axis_name="c", subcore_axis_name="s",
        num_cores=_NC, num_subcores=_NS)


def _sc_degree(col2, n_nodes):
    ch = 64
    ncw = _EPAD // (_NW * ch)
    nblk = 8
    nb = ncw // nblk
    rpt = 8 * (n_nodes // (8 * _NS))
    tail = n_nodes - rpt * _NS
    tbase = rpt * _NS

    @functools.partial(
        pl.kernel,
        out_type=jax.ShapeDtypeStruct((_NC, n_nodes, 16), jnp.float32),
        mesh=_mesh(),
        scratch_types=[
            pltpu.VMEM_SHARED((n_nodes + 8, 16), jnp.float32),
            pltpu.VMEM((nblk, ch), jnp.int32),
            pltpu.VMEM((4, ch, 16), jnp.float32),
            pltpu.VMEM((rpt, 16), jnp.float32),
            pltpu.VMEM((tail, 16), jnp.float32),
            pltpu.SemaphoreType.DMA,
            pltpu.SemaphoreType.DMA,
            pltpu.SemaphoreType.DMA,
            pltpu.SemaphoreType.DMA,
        ],
    )
    def k(col_hbm, hist_hbm, acc, ci8, ones_v, buf_v, tail_v,
          ds0, ds1, ds2, ds3):
        dsem = [ds0, ds1, ds2, ds3]
        c = lax.axis_index("c")
        s = lax.axis_index("s")
        w = s * _NC + c
        cb = w * ncw

        def fill_ones(sl):
            def body(i, _):
                ones_v[sl, i] = jnp.full((16,), 1.0, jnp.float32)
                return 0

            lax.fori_loop(0, ch, body, 0)

        for sl in range(4):
            fill_ones(sl)

        def zbody(i, _):
            buf_v[i] = jnp.zeros((16,), jnp.float32)
            return 0

        lax.fori_loop(0, rpt, zbody, 0)
        pltpu.sync_copy(buf_v, acc.at[pl.ds(s * rpt, rpt)])

        @pl.when(s == _NS - 1)
        def _():
            pltpu.sync_copy(buf_v.at[pl.ds(0, tail + 8)],
                            acc.at[pl.ds(tbase, tail + 8)])

        plsc.subcore_barrier()

        def blk(j, _):
            pltpu.sync_copy(col_hbm.at[pl.ds(cb + j * nblk, nblk)], ci8)
            hs = [None] * 4
            for b in range(nblk):
                sl = b % 4
                if hs[sl] is not None:
                    hs[sl].wait()
                hs[sl] = pltpu.async_copy(
                    ones_v.at[sl], acc.at[ci8.at[b]], dsem[sl], add=True)
            for sl in range(4):
                hs[sl].wait()
            return 0

        lax.fori_loop(0, nb, blk, 0)
        plsc.subcore_barrier()

        pltpu.sync_copy(acc.at[pl.ds(s * rpt, rpt)], buf_v)
        pltpu.sync_copy(buf_v, hist_hbm.at[c, pl.ds(s * rpt, rpt)])

        @pl.when(s == _NS - 1)
        def _():
            pltpu.sync_copy(acc.at[pl.ds(tbase, tail)], tail_v)
            pltpu.sync_copy(tail_v, hist_hbm.at[c, pl.ds(tbase, tail)])

    return k(col2.reshape(-1, ch))


def _sc_scatter(g, row2, col2, n_nodes, d):
    rpt = 8 * (n_nodes // (8 * _NS))
    tail = n_nodes - rpt * _NS
    tbase = rpt * _NS
    wb = 48
    nwb = rpt // wb

    @functools.partial(
        pl.kernel,
        out_type=jax.ShapeDtypeStruct((_NC, n_nodes, d), jnp.float32),
        mesh=_mesh(),
        scratch_types=[
            pltpu.VMEM_SHARED((n_nodes + 8, d), jnp.float32),
            pltpu.VMEM((_CH,), jnp.int32),
            pltpu.VMEM((_CH,), jnp.int32),
            pltpu.VMEM((_CH, d), jnp.float32),
            pltpu.VMEM((wb, d), jnp.float32),
            pltpu.VMEM((tail, d), jnp.float32),
        ],
    )
    def k(g_hbm, row_hbm, col_hbm, out_hbm, acc, ri_v, ci_v, rows_v,
          buf_a, tail_v):
        c = lax.axis_index("c")
        s = lax.axis_index("s")
        w = s * _NC + c
        cb = w * _NCW

        def zrow(i, _):
            for j in range(d // 16):
                buf_a[i, pl.ds(j * 16, 16)] = jnp.zeros((16,), jnp.float32)
            return 0

        lax.fori_loop(0, wb, zrow, 0)
        for j in range(nwb):
            pltpu.sync_copy(buf_a, acc.at[pl.ds(s * rpt + j * wb, wb)])

        @pl.when(s == _NS - 1)
        def _():
            pltpu.sync_copy(buf_a.at[pl.ds(0, tail + 8)],
                            acc.at[pl.ds(tbase, tail + 8)])

        plsc.subcore_barrier()

        eb = cb * _CH

        def chunk(ch, _):
            pltpu.sync_copy(row_hbm.at[pl.ds(eb + ch * _CH, _CH)], ri_v)
            pltpu.sync_copy(col_hbm.at[pl.ds(eb + ch * _CH, _CH)], ci_v)
            pltpu.sync_copy(g_hbm.at[ri_v], rows_v)
            pltpu.sync_copy(rows_v, acc.at[ci_v], add=True)
            return 0

        lax.fori_loop(0, _NCW, chunk, 0)
        plsc.subcore_barrier()

        for j in range(nwb):
            pltpu.sync_copy(acc.at[pl.ds(s * rpt + j * wb, wb)], buf_a)
            pltpu.sync_copy(buf_a, out_hbm.at[c, pl.ds(s * rpt + j * wb, wb)])

        @pl.when(s == _NS - 1)
        def _():
            pltpu.sync_copy(acc.at[pl.ds(tbase, tail)], tail_v)
            pltpu.sync_copy(tail_v, out_hbm.at[c, pl.ds(tbase, tail)])

    return k(g, row2.reshape(-1), col2.reshape(-1))


def _tc_transform(x, w, hist):
    n, d_in = x.shape
    d_out = w.shape[1]
    blk = 1000

    def body(x_ref, w_ref, h_ref, g_ref):
        deg = (h_ref[0] + h_ref[1])[:, 0:1] + 1.0
        dinv = lax.rsqrt(deg)
        h = jnp.dot(x_ref[...], w_ref[...], preferred_element_type=jnp.float32)
        g_ref[...] = h * dinv

    return pl.pallas_call(
        body,
        grid=(n // blk,),
        in_specs=[
            pl.BlockSpec((blk, d_in), lambda i: (i, 0)),
            pl.BlockSpec((d_in, d_out), lambda i: (0, 0)),
            pl.BlockSpec((2, blk, 16), lambda i: (0, i, 0)),
        ],
        out_specs=pl.BlockSpec((blk, d_out), lambda i: (i, 0)),
        out_shape=jax.ShapeDtypeStruct((n, d_out), jnp.float32),
    )(x, w, hist)


def _tc_finish(accp, g, hist, b):
    n, d = g.shape
    blk = 1000
    b2 = b.reshape(1, d)

    def body(a_ref, g_ref, h_ref, b_ref, o_ref):
        deg = (h_ref[0] + h_ref[1])[:, 0:1] + 1.0
        dinv = lax.rsqrt(deg)
        s = a_ref[0] + a_ref[1] + g_ref[...]
        o_ref[...] = s * dinv + b_ref[...]

    return pl.pallas_call(
        body,
        grid=(n // blk,),
        in_specs=[
            pl.BlockSpec((2, blk, d), lambda i: (0, i, 0)),
            pl.BlockSpec((blk, d), lambda i: (i, 0)),
            pl.BlockSpec((2, blk, 16), lambda i: (0, i, 0)),
            pl.BlockSpec((1, d), lambda i: (0, 0)),
        ],
        out_specs=pl.BlockSpec((blk, d), lambda i: (i, 0)),
        out_shape=jax.ShapeDtypeStruct((n, d), jnp.float32),
    )(accp, g, hist, b2)


def kernel(x, edge_index, W, b):
    n = x.shape[0]
    d = W.shape[1]
    e = edge_index.shape[1]
    npad = _EPAD - e
    row2 = jnp.concatenate(
        [edge_index[0], jnp.zeros((npad,), jnp.int32)]).reshape(-1, _CH)
    col2 = jnp.concatenate(
        [edge_index[1], jnp.full((npad,), n, jnp.int32)]).reshape(-1, _CH)
    hist = _sc_degree(col2, n)
    g = _tc_transform(x, W, hist)
    accp = _sc_scatter(g, row2, col2, n, d)
    return _tc_finish(accp, g, hist, b)


# speedup vs baseline: 1.0005x; 1.0005x over previous
import functools

import jax
import jax.numpy as jnp
from jax import lax
from jax.experimental import pallas as pl
from jax.experimental.pallas import tpu as pltpu
from jax.experimental.pallas import tpu_sc as plsc

_NC = 2
_NS = 16
_NW = _NC * _NS
_CH = 128
_NCW = 80
_NBLK = 8
_NB = _NCW // _NBLK
_IBLK = 16
_NIB = _NCW // _IBLK
_EPAD = _NW * _NCW * _CH


def _mesh():
    return plsc.VectorSubcoreMesh(
        core_axis_name="c", subcore_axis_name="s",
        num_cores=_NC, num_subcores=_NS)


def _sc_degree(col2, n_nodes):
    ch = 64
    ncw = _EPAD // (_NW * ch)
    nblk = 8
    nb = ncw // nblk
    rpt = 8 * (n_nodes // (8 * _NS))
    tail = n_nodes - rpt * _NS
    tbase = rpt * _NS

    @functools.partial(
        pl.kernel,
        out_type=jax.ShapeDtypeStruct((_NC, n_nodes, 16), jnp.float32),
        mesh=_mesh(),
        scratch_types=[
            pltpu.VMEM_SHARED((n_nodes + 128, 16), jnp.float32),
            pltpu.VMEM((nblk, ch), jnp.int32),
            pltpu.VMEM((4, ch, 16), jnp.float32),
            pltpu.VMEM((rpt, 16), jnp.float32),
            pltpu.VMEM((tail, 16), jnp.float32),
            pltpu.SemaphoreType.DMA,
            pltpu.SemaphoreType.DMA,
            pltpu.SemaphoreType.DMA,
            pltpu.SemaphoreType.DMA,
        ],
    )
    def k(col_hbm, hist_hbm, acc, ci8, ones_v, buf_v, tail_v,
          ds0, ds1, ds2, ds3):
        dsem = [ds0, ds1, ds2, ds3]
        c = lax.axis_index("c")
        s = lax.axis_index("s")
        w = s * _NC + c
        cb = w * ncw

        def fill_ones(sl):
            def body(i, _):
                ones_v[sl, i] = jnp.full((16,), 1.0, jnp.float32)
                return 0

            lax.fori_loop(0, ch, body, 0)

        for sl in range(4):
            fill_ones(sl)

        def zbody(i, _):
            buf_v[i] = jnp.zeros((16,), jnp.float32)
            return 0

        lax.fori_loop(0, rpt, zbody, 0)
        pltpu.sync_copy(buf_v, acc.at[pl.ds(s * rpt, rpt)])

        @pl.when(s == _NS - 1)
        def _():
            pltpu.sync_copy(buf_v.at[pl.ds(0, tail + 128)],
                            acc.at[pl.ds(tbase, tail + 128)])

        plsc.subcore_barrier()

        def blk(j, _):
            pltpu.sync_copy(col_hbm.at[pl.ds(cb + j * nblk, nblk)], ci8)
            hs = [None] * 4
            for b in range(nblk):
                sl = b % 4
                if hs[sl] is not None:
                    hs[sl].wait()
                hs[sl] = pltpu.async_copy(
                    ones_v.at[sl], acc.at[ci8.at[b]], dsem[sl], add=True)
            for sl in range(4):
                hs[sl].wait()
            return 0

        lax.fori_loop(0, nb, blk, 0)
        plsc.subcore_barrier()

        pltpu.sync_copy(acc.at[pl.ds(s * rpt, rpt)], buf_v)
        pltpu.sync_copy(buf_v, hist_hbm.at[c, pl.ds(s * rpt, rpt)])

        @pl.when(s == _NS - 1)
        def _():
            pltpu.sync_copy(acc.at[pl.ds(tbase, tail)], tail_v)
            pltpu.sync_copy(tail_v, hist_hbm.at[c, pl.ds(tbase, tail)])

    return k(col2.reshape(-1, ch))


def _sc_scatter(g, row2, col2, n_nodes, d):
    rpt = 8 * (n_nodes // (8 * _NS))
    tail = n_nodes - rpt * _NS
    tbase = rpt * _NS
    wb = 48
    nwb = rpt // wb

    @functools.partial(
        pl.kernel,
        out_type=jax.ShapeDtypeStruct((_NC, n_nodes, d), jnp.float32),
        mesh=_mesh(),
        scratch_types=[
            pltpu.VMEM_SHARED((n_nodes + 128, d), jnp.float32),
            pltpu.VMEM((_CH,), jnp.int32),
            pltpu.VMEM((_CH,), jnp.int32),
            pltpu.VMEM((_CH, d), jnp.float32),
            pltpu.VMEM((wb, d), jnp.float32),
            pltpu.VMEM((tail, d), jnp.float32),
        ],
    )
    def k(g_hbm, row_hbm, col_hbm, out_hbm, acc, ri_v, ci_v, rows_v,
          buf_a, tail_v):
        c = lax.axis_index("c")
        s = lax.axis_index("s")
        w = s * _NC + c
        cb = w * _NCW

        def zrow(i, _):
            for j in range(d // 16):
                buf_a[i, pl.ds(j * 16, 16)] = jnp.zeros((16,), jnp.float32)
            return 0

        lax.fori_loop(0, wb, zrow, 0)
        for j in range(nwb):
            pltpu.sync_copy(buf_a, acc.at[pl.ds(s * rpt + j * wb, wb)])

        @pl.when(s == _NS - 1)
        def _():
            for j in range(3):
                pltpu.sync_copy(buf_a, acc.at[pl.ds(tbase + j * wb, wb)])

        plsc.subcore_barrier()

        eb = cb * _CH

        def chunk(ch, _):
            pltpu.sync_copy(row_hbm.at[pl.ds(eb + ch * _CH, _CH)], ri_v)
            pltpu.sync_copy(col_hbm.at[pl.ds(eb + ch * _CH, _CH)], ci_v)
            pltpu.sync_copy(g_hbm.at[ri_v], rows_v)
            pltpu.sync_copy(rows_v, acc.at[ci_v], add=True)
            return 0

        lax.fori_loop(0, _NCW, chunk, 0)
        plsc.subcore_barrier()

        for j in range(nwb):
            pltpu.sync_copy(acc.at[pl.ds(s * rpt + j * wb, wb)], buf_a)
            pltpu.sync_copy(buf_a, out_hbm.at[c, pl.ds(s * rpt + j * wb, wb)])

        @pl.when(s == _NS - 1)
        def _():
            pltpu.sync_copy(acc.at[pl.ds(tbase, tail)], tail_v)
            pltpu.sync_copy(tail_v, out_hbm.at[c, pl.ds(tbase, tail)])

    return k(g, row2.reshape(-1), col2.reshape(-1))


def _tc_transform(x, w, hist):
    n, d_in = x.shape
    d_out = w.shape[1]
    blk = 1000

    def body(x_ref, w_ref, h_ref, g_ref):
        deg = (h_ref[0] + h_ref[1])[:, 0:1] + 1.0
        dinv = lax.rsqrt(deg)
        h = jnp.dot(x_ref[...], w_ref[...], preferred_element_type=jnp.float32)
        g_ref[...] = h * dinv

    return pl.pallas_call(
        body,
        grid=(n // blk,),
        in_specs=[
            pl.BlockSpec((blk, d_in), lambda i: (i, 0)),
            pl.BlockSpec((d_in, d_out), lambda i: (0, 0)),
            pl.BlockSpec((2, blk, 16), lambda i: (0, i, 0)),
        ],
        out_specs=pl.BlockSpec((blk, d_out), lambda i: (i, 0)),
        out_shape=jax.ShapeDtypeStruct((n, d_out), jnp.float32),
    )(x, w, hist)


def _tc_finish(accp, g, hist, b):
    n, d = g.shape
    blk = 1000
    b2 = b.reshape(1, d)

    def body(a_ref, g_ref, h_ref, b_ref, o_ref):
        deg = (h_ref[0] + h_ref[1])[:, 0:1] + 1.0
        dinv = lax.rsqrt(deg)
        s = a_ref[0] + a_ref[1] + g_ref[...]
        o_ref[...] = s * dinv + b_ref[...]

    return pl.pallas_call(
        body,
        grid=(n // blk,),
        in_specs=[
            pl.BlockSpec((2, blk, d), lambda i: (0, i, 0)),
            pl.BlockSpec((blk, d), lambda i: (i, 0)),
            pl.BlockSpec((2, blk, 16), lambda i: (0, i, 0)),
            pl.BlockSpec((1, d), lambda i: (0, 0)),
        ],
        out_specs=pl.BlockSpec((blk, d), lambda i: (i, 0)),
        out_shape=jax.ShapeDtypeStruct((n, d), jnp.float32),
    )(accp, g, hist, b2)


def kernel(x, edge_index, W, b):
    n = x.shape[0]
    d = W.shape[1]
    e = edge_index.shape[1]
    npad = _EPAD - e
    row2 = jnp.concatenate(
        [edge_index[0], jnp.zeros((npad,), jnp.int32)]).reshape(-1, _CH)
    dummy = n + (jnp.arange(npad, dtype=jnp.int32) % 128)
    col2 = jnp.concatenate([edge_index[1], dummy]).reshape(-1, _CH)
    hist = _sc_degree(col2, n)
    g = _tc_transform(x, W, hist)
    accp = _sc_scatter(g, row2, col2, n, d)
    return _tc_finish(accp, g, hist, b)


# speedup vs baseline: 2.0203x; 2.0193x over previous
import functools

import jax
import jax.numpy as jnp
from jax import lax
from jax.experimental import pallas as pl
from jax.experimental.pallas import tpu as pltpu
from jax.experimental.pallas import tpu_sc as plsc

_NC = 2
_NS = 16
_NW = _NC * _NS
_CH = 128
_NCW = 80
_NBLK = 8
_NB = _NCW // _NBLK
_IBLK = 16
_NIB = _NCW // _IBLK
_EPAD = _NW * _NCW * _CH


def _mesh():
    return plsc.VectorSubcoreMesh(
        core_axis_name="c", subcore_axis_name="s",
        num_cores=_NC, num_subcores=_NS)


def _sc_degree(col2, n_nodes):
    ch = 64
    ncw = _EPAD // (_NW * ch)
    nblk = 8
    nb = ncw // nblk
    rpt = 8 * (n_nodes // (8 * _NS))
    tail = n_nodes - rpt * _NS
    tbase = rpt * _NS

    @functools.partial(
        pl.kernel,
        out_type=jax.ShapeDtypeStruct((_NC, n_nodes, 16), jnp.float32),
        mesh=_mesh(),
        scratch_types=[
            pltpu.VMEM_SHARED((n_nodes + 128, 16), jnp.float32),
            pltpu.VMEM((nblk, ch), jnp.int32),
            pltpu.VMEM((4, ch, 16), jnp.float32),
            pltpu.VMEM((rpt, 16), jnp.float32),
            pltpu.VMEM((tail, 16), jnp.float32),
            pltpu.SemaphoreType.DMA,
            pltpu.SemaphoreType.DMA,
            pltpu.SemaphoreType.DMA,
            pltpu.SemaphoreType.DMA,
        ],
    )
    def k(col_hbm, hist_hbm, acc, ci8, ones_v, buf_v, tail_v,
          ds0, ds1, ds2, ds3):
        dsem = [ds0, ds1, ds2, ds3]
        c = lax.axis_index("c")
        s = lax.axis_index("s")
        w = s * _NC + c
        cb = w * ncw

        def fill_ones(sl):
            def body(i, _):
                ones_v[sl, i] = jnp.full((16,), 1.0, jnp.float32)
                return 0

            lax.fori_loop(0, ch, body, 0)

        for sl in range(4):
            fill_ones(sl)

        def zbody(i, _):
            buf_v[i] = jnp.zeros((16,), jnp.float32)
            return 0

        lax.fori_loop(0, rpt, zbody, 0)
        pltpu.sync_copy(buf_v, acc.at[pl.ds(s * rpt, rpt)])

        @pl.when(s == _NS - 1)
        def _():
            pltpu.sync_copy(buf_v.at[pl.ds(0, tail + 128)],
                            acc.at[pl.ds(tbase, tail + 128)])

        plsc.subcore_barrier()

        def blk(j, _):
            pltpu.sync_copy(col_hbm.at[pl.ds(cb + j * nblk, nblk)], ci8)
            hs = [None] * 4
            for b in range(nblk):
                sl = b % 4
                if hs[sl] is not None:
                    hs[sl].wait()
                hs[sl] = pltpu.async_copy(
                    ones_v.at[sl], acc.at[ci8.at[b]], dsem[sl], add=True)
            for sl in range(4):
                hs[sl].wait()
            return 0

        lax.fori_loop(0, nb, blk, 0)
        plsc.subcore_barrier()

        pltpu.sync_copy(acc.at[pl.ds(s * rpt, rpt)], buf_v)
        pltpu.sync_copy(buf_v, hist_hbm.at[c, pl.ds(s * rpt, rpt)])

        @pl.when(s == _NS - 1)
        def _():
            pltpu.sync_copy(acc.at[pl.ds(tbase, tail)], tail_v)
            pltpu.sync_copy(tail_v, hist_hbm.at[c, pl.ds(tbase, tail)])

    return k(col2.reshape(-1, ch))


def _sc_scatter(g, row2, col2, n_nodes, d):
    rpt = 8 * (n_nodes // (8 * _NS))
    tail = n_nodes - rpt * _NS
    tbase = rpt * _NS
    wb = 48
    nwb = rpt // wb

    @functools.partial(
        pl.kernel,
        out_type=jax.ShapeDtypeStruct((_NC, n_nodes, d), jnp.float32),
        mesh=_mesh(),
        scratch_types=[
            pltpu.VMEM_SHARED((n_nodes + 128, d), jnp.float32),
            pltpu.VMEM((_CH,), jnp.int32),
            pltpu.VMEM((_CH,), jnp.int32),
            pltpu.VMEM((_CH, d), jnp.float32),
            pltpu.VMEM((wb, d), jnp.float32),
            pltpu.VMEM((tail, d), jnp.float32),
        ],
    )
    def k(g_hbm, row_hbm, col_hbm, out_hbm, acc, ri_v, ci_v, rows_v,
          buf_a, tail_v):
        c = lax.axis_index("c")
        s = lax.axis_index("s")
        w = s * _NC + c
        cb = w * _NCW

        def zrow(i, _):
            for j in range(d // 16):
                buf_a[i, pl.ds(j * 16, 16)] = jnp.zeros((16,), jnp.float32)
            return 0

        lax.fori_loop(0, wb, zrow, 0)
        for j in range(nwb):
            pltpu.sync_copy(buf_a, acc.at[pl.ds(s * rpt + j * wb, wb)])

        @pl.when(s == _NS - 1)
        def _():
            for j in range(3):
                pltpu.sync_copy(buf_a, acc.at[pl.ds(tbase + j * wb, wb)])

        plsc.subcore_barrier()

        eb = cb * _CH

        def chunk(ch, _):
            pltpu.sync_copy(row_hbm.at[pl.ds(eb + ch * _CH, _CH)], ri_v)
            pltpu.sync_copy(col_hbm.at[pl.ds(eb + ch * _CH, _CH)], ci_v)
            pltpu.sync_copy(g_hbm.at[ri_v], rows_v)
            pltpu.sync_copy(rows_v, acc.at[ci_v], add=True)
            return 0

        lax.fori_loop(0, _NCW, chunk, 0)
        plsc.subcore_barrier()

        for j in range(nwb):
            pltpu.sync_copy(acc.at[pl.ds(s * rpt + j * wb, wb)], buf_a)
            pltpu.sync_copy(buf_a, out_hbm.at[c, pl.ds(s * rpt + j * wb, wb)])

        @pl.when(s == _NS - 1)
        def _():
            pltpu.sync_copy(acc.at[pl.ds(tbase, tail)], tail_v)
            pltpu.sync_copy(tail_v, out_hbm.at[c, pl.ds(tbase, tail)])

    return k(g, row2.reshape(-1), col2.reshape(-1))


def _tc_transform(x, w, hist):
    n, d_in = x.shape
    d_out = w.shape[1]
    blk = 1000

    def body(x_ref, w_ref, h_ref, g_ref):
        deg = (h_ref[0] + h_ref[1])[:, 0:1] + 1.0
        dinv = lax.rsqrt(deg)
        h = jnp.dot(x_ref[...], w_ref[...], preferred_element_type=jnp.float32)
        g_ref[...] = h * dinv

    return pl.pallas_call(
        body,
        grid=(n // blk,),
        in_specs=[
            pl.BlockSpec((blk, d_in), lambda i: (i, 0)),
            pl.BlockSpec((d_in, d_out), lambda i: (0, 0)),
            pl.BlockSpec((2, blk, 16), lambda i: (0, i, 0)),
        ],
        out_specs=pl.BlockSpec((blk, d_out), lambda i: (i, 0)),
        out_shape=jax.ShapeDtypeStruct((n, d_out), jnp.float32),
    )(x, w, hist)


def _tc_finish(accp, g, hist, b):
    n, d = g.shape
    blk = 1000
    b2 = b.reshape(1, d)

    def body(a_ref, g_ref, h_ref, b_ref, o_ref):
        deg = (h_ref[0] + h_ref[1])[:, 0:1] + 1.0
        dinv = lax.rsqrt(deg)
        s = a_ref[0] + a_ref[1] + g_ref[...]
        o_ref[...] = s * dinv + b_ref[...]

    return pl.pallas_call(
        body,
        grid=(n // blk,),
        in_specs=[
            pl.BlockSpec((2, blk, d), lambda i: (0, i, 0)),
            pl.BlockSpec((blk, d), lambda i: (i, 0)),
            pl.BlockSpec((2, blk, 16), lambda i: (0, i, 0)),
            pl.BlockSpec((1, d), lambda i: (0, 0)),
        ],
        out_specs=pl.BlockSpec((blk, d), lambda i: (i, 0)),
        out_shape=jax.ShapeDtypeStruct((n, d), jnp.float32),
    )(accp, g, hist, b2)


def kernel(x, edge_index, W, b):
    n = x.shape[0]
    d = W.shape[1]
    e = edge_index.shape[1]
    npad = _EPAD - e
    spread = jnp.arange(npad, dtype=jnp.int32)
    row2 = jnp.concatenate([edge_index[0], spread % n]).reshape(-1, _CH)
    col2 = jnp.concatenate([edge_index[1], n + spread % 128]).reshape(-1, _CH)
    hist = _sc_degree(col2, n)
    g = _tc_transform(x, W, hist)
    accp = _sc_scatter(g, row2, col2, n, d)
    return _tc_finish(accp, g, hist, b)
